# baseline (speedup 1.0000x reference)
import functools

import jax
import jax.numpy as jnp
from jax import lax
from jax.experimental import pallas as pl
from jax.experimental.pallas import tpu as pltpu
from jax.experimental.pallas import tpu_sc as plsc

B = 16384
TWO_B = 2 * B
N = 100000
D = 64
TD = 32
NC = 2
NS = 16
NW = NC * NS

_PREC = lax.Precision.HIGHEST
_MESH = plsc.VectorSubcoreMesh(core_axis_name="c", subcore_axis_name="s")
_SC_PARAMS = pltpu.CompilerParams(use_tc_tiling_on_sc=False)

MASK_PAD = 100096


def _dotg(a, b):
  return lax.dot_general(a, b, (((0,), (0,)), ((), ())),
                         precision=_PREC, preferred_element_type=jnp.float32)


def _mm(a, b):
  return lax.dot_general(a, b, (((1,), (0,)), ((), ())),
                         precision=_PREC, preferred_element_type=jnp.float32)


def _k1_body(ids2d, memory, last_update, mem_rows, lu2d, idx_v, rbuf, lubuf):
  c = lax.axis_index("c")
  s = lax.axis_index("s")
  w = c * NS + s
  pltpu.sync_copy(ids2d.at[pl.ds(w * 8, 8), :], idx_v)

  def step(j, _):
    pltpu.sync_copy(memory.at[idx_v.at[j]], rbuf.at[j])
    pltpu.sync_copy(last_update.at[idx_v.at[j]], lubuf.at[j])
    pltpu.sync_copy(rbuf.at[j], mem_rows.at[pl.ds(w * 1024 + j * 128, 128), :])
    return 0

  lax.fori_loop(0, 8, step, 0)
  pltpu.sync_copy(lubuf, lu2d.at[pl.ds(w * 8, 8), :])


_k1 = pl.kernel(
    _k1_body,
    out_type=(
        jax.ShapeDtypeStruct((TWO_B, D), jnp.float32),
        jax.ShapeDtypeStruct((TWO_B // 128, 128), jnp.int32),
    ),
    mesh=_MESH,
    compiler_params=_SC_PARAMS,
    scratch_types=[
        pltpu.VMEM((8, 128), jnp.int32),
        pltpu.VMEM((8, 128, D), jnp.float32),
        pltpu.VMEM((8, 128), jnp.int32),
    ],
)


def _k2_body(m1, m2, raw, t3, lu3, wrow, btime, wtr, wtz, wtn, p_out):
  trel = (t3[0] - lu3[0]).astype(jnp.float32)
  tenc = jnp.cos(_dotg(trel, wrow[...]) + btime[...])
  a, b, r = m1[...], m2[...], raw[...]
  for g, wt in enumerate((wtr, wtz, wtn)):
    p_out[g] = (_mm(a, wt[0:64]) + _mm(b, wt[64:128])
                + _mm(r, wt[128:192]) + _mm(tenc, wt[192:224]))


def _k2(mem_rows, raw_msg, t3, lu3, wrow, btime, wtr, wtz, wtn):
  RB = 2048
  grid = (B // RB, 2)
  return pl.pallas_call(
      _k2_body,
      grid=grid,
      in_specs=[
          pl.BlockSpec((RB, D), lambda i, h: (h * 8 + i, 0)),
          pl.BlockSpec((RB, D), lambda i, h: ((1 - h) * 8 + i, 0)),
          pl.BlockSpec((RB, D), lambda i, h: (i, 0)),
          pl.BlockSpec((1, 1, RB), lambda i, h: (i, 0, 0)),
          pl.BlockSpec((1, 1, RB), lambda i, h: (h * 8 + i, 0, 0)),
          pl.BlockSpec((1, TD), lambda i, h: (0, 0)),
          pl.BlockSpec((1, TD), lambda i, h: (0, 0)),
          pl.BlockSpec((224, D), lambda i, h: (0, 0)),
          pl.BlockSpec((224, D), lambda i, h: (0, 0)),
          pl.BlockSpec((224, D), lambda i, h: (0, 0)),
      ],
      out_specs=pl.BlockSpec((3, RB, D), lambda i, h: (0, h * 8 + i, 0)),
      out_shape=jax.ShapeDtypeStruct((3, TWO_B, D), jnp.float32),
  )(mem_rows, mem_rows, raw_msg, t3, lu3, wrow, btime, wtr, wtz, wtn)


def _k3_body(ids2d, p_parts, zrow, zblk, s_out, cnt2d,
             table, cnt_tbl, ids_v, pbuf, obuf, zbuf, z1, ones_v, cbuf):
  c = lax.axis_index("c")
  s = lax.axis_index("s")
  r0 = s * 2048
  pltpu.sync_copy(ids2d.at[pl.ds(s * 16, 16), :], ids_v)
  pltpu.sync_copy(zblk, zbuf)
  pltpu.sync_copy(zrow.at[0], z1)
  pltpu.sync_copy(zrow.at[1], ones_v)

  @pl.when(c == 0)
  def _counts():
    def zstep(j, _):
      pltpu.sync_copy(z1, cnt_tbl.at[ids_v.at[j]])
      return 0
    lax.fori_loop(0, 16, zstep, 0)
    plsc.subcore_barrier()

    def astep(j, _):
      pltpu.sync_copy(ones_v, cnt_tbl.at[ids_v.at[j]], add=True)
      return 0
    lax.fori_loop(0, 16, astep, 0)
    plsc.subcore_barrier()

    def gstep(j, _):
      pltpu.sync_copy(cnt_tbl.at[ids_v.at[j]], cbuf.at[j])
      return 0
    lax.fori_loop(0, 16, gstep, 0)
    pltpu.sync_copy(cbuf, cnt2d.at[pl.ds(s * 16, 16), :])

  for q in range(12):
    k = c * 12 + q
    part = k // 8
    col0 = (k % 8) * 8
    pltpu.sync_copy(p_parts.at[part, pl.ds(r0, 2048), pl.ds(col0, 8)], pbuf)

    def zstep(j, _):
      pltpu.sync_copy(zbuf, table.at[ids_v.at[j]])
      return 0
    lax.fori_loop(0, 16, zstep, 0)
    plsc.subcore_barrier()

    def astep(j, _):
      pltpu.sync_copy(pbuf.at[pl.ds(j * 128, 128), :], table.at[ids_v.at[j]],
                      add=True)
      return 0
    lax.fori_loop(0, 16, astep, 0)
    plsc.subcore_barrier()

    def gstep(j, _):
      pltpu.sync_copy(table.at[ids_v.at[j]], obuf.at[pl.ds(j * 128, 128), :])
      return 0
    lax.fori_loop(0, 16, gstep, 0)
    pltpu.sync_copy(obuf, s_out.at[part, pl.ds(r0, 2048), pl.ds(col0, 8)])
    plsc.subcore_barrier()


_k3 = pl.kernel(
    _k3_body,
    out_type=(
        jax.ShapeDtypeStruct((3, TWO_B, D), jnp.float32),
        jax.ShapeDtypeStruct((TWO_B // 128, 128), jnp.float32),
    ),
    mesh=_MESH,
    compiler_params=_SC_PARAMS,
    scratch_types=[
        pltpu.VMEM_SHARED((N, 8), jnp.float32),
        pltpu.VMEM_SHARED((N,), jnp.float32),
        pltpu.VMEM((16, 128), jnp.int32),
        pltpu.VMEM((2048, 8), jnp.float32),
        pltpu.VMEM((2048, 8), jnp.float32),
        pltpu.VMEM((128, 8), jnp.float32),
        pltpu.VMEM((128,), jnp.float32),
        pltpu.VMEM((128,), jnp.float32),
        pltpu.VMEM((16, 128), jnp.float32),
    ],
)


def _k4_body(s_parts, cnt3, mem_blk, whr, whz, whn,
             bir, biz, bin_, bhr, bhz, bhn, h_out):
  ones = jnp.ones((1, D), jnp.float32)
  inv = _dotg(1.0 / cnt3[0], ones)
  h = mem_blk[...]
  gi_r = s_parts[0] * inv + bir[...]
  gi_z = s_parts[1] * inv + biz[...]
  gi_n = s_parts[2] * inv + bin_[...]
  gh_r = _mm(h, whr[...]) + bhr[...]
  gh_z = _mm(h, whz[...]) + bhz[...]
  gh_n = _mm(h, whn[...]) + bhn[...]
  r = 1.0 / (1.0 + jnp.exp(-(gi_r + gh_r)))
  z = 1.0 / (1.0 + jnp.exp(-(gi_z + gh_z)))
  n = jnp.tanh(gi_n + r * gh_n)
  h_out[...] = (1.0 - z) * n + z * h


def _k4(s_parts, cnt3, mem_rows, whr, whz, whn, bir, biz, bin_, bhr, bhz, bhn):
  RB = 2048
  wspec = pl.BlockSpec((D, D), lambda i: (0, 0))
  bspec = pl.BlockSpec((1, D), lambda i: (0, 0))
  return pl.pallas_call(
      _k4_body,
      grid=(TWO_B // RB,),
      in_specs=[
          pl.BlockSpec((3, RB, D), lambda i: (0, i, 0)),
          pl.BlockSpec((1, 1, RB), lambda i: (i, 0, 0)),
          pl.BlockSpec((RB, D), lambda i: (i, 0)),
          wspec, wspec, wspec, bspec, bspec, bspec, bspec, bspec, bspec,
      ],
      out_specs=pl.BlockSpec((RB, D), lambda i: (i, 0)),
      out_shape=jax.ShapeDtypeStruct((TWO_B, D), jnp.float32),
  )(s_parts, cnt3, mem_rows, whr, whz, whn, bir, biz, bin_, bhr, bhz, bhn)


def _k5_body(ids2d, h_new, zrow, zmask_in, delta, mask0, mask1,
             idx_v, hbuf, zmask, ones_v):
  c = lax.axis_index("c")
  s = lax.axis_index("s")
  w = c * NS + s
  pltpu.sync_copy(zmask_in, zmask)
  pltpu.sync_copy(zrow.at[1], ones_v)

  @pl.when(c == 0)
  def _z0():
    pltpu.sync_copy(zmask, mask0.at[pl.ds(s * 6256, 6256)])

  @pl.when(c == 1)
  def _z1():
    pltpu.sync_copy(zmask, mask1.at[pl.ds(s * 6256, 6256)])

  plsc.subcore_barrier()

  pltpu.sync_copy(ids2d.at[pl.ds(w * 8, 8), :], idx_v)
  pltpu.sync_copy(h_new.at[pl.ds(w * 1024, 1024), :], hbuf)

  def step(j, _):
    @pl.when(c == 0)
    def _m0():
      pltpu.sync_copy(ones_v, mask0.at[idx_v.at[j]])

    @pl.when(c == 1)
    def _m1():
      pltpu.sync_copy(ones_v, mask1.at[idx_v.at[j]])

    pltpu.sync_copy(hbuf.at[pl.ds(j * 128, 128), :], delta.at[idx_v.at[j]])
    return 0

  lax.fori_loop(0, 8, step, 0)


_k5 = pl.kernel(
    _k5_body,
    out_type=(
        jax.ShapeDtypeStruct((N, D), jnp.float32),
        jax.ShapeDtypeStruct((MASK_PAD,), jnp.float32),
        jax.ShapeDtypeStruct((MASK_PAD,), jnp.float32),
    ),
    mesh=_MESH,
    compiler_params=_SC_PARAMS,
    scratch_types=[
        pltpu.VMEM((8, 128), jnp.int32),
        pltpu.VMEM((1024, D), jnp.float32),
        pltpu.VMEM((6256,), jnp.float32),
        pltpu.VMEM((128,), jnp.float32),
    ],
)


def _k6_body(memory, delta, m0, m1, out):
  ones = jnp.ones((1, D), jnp.float32)
  touched = _dotg(m0[0] + m1[0], ones)
  out[...] = jnp.where(touched > 0.0, delta[...], memory[...])


def _k6(memory, delta, m0_3, m1_3):
  RB = 10000
  return pl.pallas_call(
      _k6_body,
      grid=(N // RB,),
      in_specs=[
          pl.BlockSpec((RB, D), lambda i: (i, 0)),
          pl.BlockSpec((RB, D), lambda i: (i, 0)),
          pl.BlockSpec((1, 1, RB), lambda i: (i, 0, 0)),
          pl.BlockSpec((1, 1, RB), lambda i: (i, 0, 0)),
      ],
      out_specs=pl.BlockSpec((RB, D), lambda i: (i, 0)),
      out_shape=jax.ShapeDtypeStruct((N, D), jnp.float32),
  )(memory, delta, m0_3, m1_3)


def kernel(src, pos_dst, neg_dst, t, raw_msg, memory, last_update,
           W_time, b_time, W_ih, W_hh, b_ih, b_hh):
  del neg_dst
  ids = jnp.concatenate([src, pos_dst]).astype(jnp.int32)
  ids2d = ids.reshape(TWO_B // 128, 128)

  mem_rows, lu2d = _k1(ids2d, memory, last_update.astype(jnp.int32))

  t3 = t.astype(jnp.int32).reshape(B // 2048, 1, 2048)
  lu3 = lu2d.reshape(TWO_B // 2048, 1, 2048)
  wrow = W_time[:, 0].reshape(1, TD)
  btime = b_time.reshape(1, TD)
  WT = W_ih.T
  wtr, wtz, wtn = WT[:, 0:D], WT[:, D:2 * D], WT[:, 2 * D:3 * D]
  p_parts = _k2(mem_rows, raw_msg, t3, lu3, wrow, btime, wtr, wtz, wtn)

  zrow = jnp.stack([jnp.zeros((128,), jnp.float32),
                    jnp.ones((128,), jnp.float32)])
  zblk = jnp.zeros((128, 8), jnp.float32)
  s_parts, cnt2d = _k3(ids2d, p_parts, zrow, zblk)

  WHT = W_hh.T
  whr, whz, whn = WHT[:, 0:D], WHT[:, D:2 * D], WHT[:, 2 * D:3 * D]
  bir = b_ih[0:D].reshape(1, D)
  biz = b_ih[D:2 * D].reshape(1, D)
  bin_ = b_ih[2 * D:3 * D].reshape(1, D)
  bhr = b_hh[0:D].reshape(1, D)
  bhz = b_hh[D:2 * D].reshape(1, D)
  bhn = b_hh[2 * D:3 * D].reshape(1, D)
  cnt3 = cnt2d.reshape(TWO_B // 2048, 1, 2048)
  h_new = _k4(s_parts, cnt3, mem_rows, whr, whz, whn,
              bir, biz, bin_, bhr, bhz, bhn)

  zmask_in = jnp.zeros((6256,), jnp.float32)
  delta, mask0, mask1 = _k5(ids2d, h_new, zrow, zmask_in)
  m0_3 = mask0[:N].reshape(N // 10000, 1, 10000)
  m1_3 = mask1[:N].reshape(N // 10000, 1, 10000)
  return _k6(memory, delta, m0_3, m1_3)


# speedup vs baseline: 1.0160x; 1.0160x over previous
import jax
import jax.numpy as jnp
from jax import lax
from jax.experimental import pallas as pl
from jax.experimental.pallas import tpu as pltpu
from jax.experimental.pallas import tpu_sc as plsc

B = 16384
TWO_B = 2 * B
N = 100000
D = 64
TD = 32
NC = 2
NS = 16
NW = NC * NS

_MESH = plsc.VectorSubcoreMesh(core_axis_name="c", subcore_axis_name="s")
_SC_PARAMS = pltpu.CompilerParams(use_tc_tiling_on_sc=False)

MASK_PAD = 100096


def _mm(a, b):
  return lax.dot_general(a, b, (((1,), (0,)), ((), ())),
                         preferred_element_type=jnp.float32)


def _k1_body(ids1d, memory, last_update, mem_rows, lu1d, idx_v, rbuf, lubuf):
  c = lax.axis_index("c")
  s = lax.axis_index("s")
  w = c * NS + s
  pltpu.sync_copy(ids1d.at[pl.ds(w * 1024, 1024)], idx_v)
  pltpu.sync_copy(memory.at[idx_v], rbuf)
  pltpu.sync_copy(last_update.at[idx_v], lubuf)
  pltpu.sync_copy(rbuf, mem_rows.at[pl.ds(w * 1024, 1024), :])
  pltpu.sync_copy(lubuf, lu1d.at[pl.ds(w * 1024, 1024)])


_k1 = pl.kernel(
    _k1_body,
    out_type=(
        jax.ShapeDtypeStruct((TWO_B, D), jnp.float32),
        jax.ShapeDtypeStruct((TWO_B,), jnp.int32),
    ),
    mesh=_MESH,
    compiler_params=_SC_PARAMS,
    scratch_types=[
        pltpu.VMEM((1024,), jnp.int32),
        pltpu.VMEM((1024, D), jnp.float32),
        pltpu.VMEM((1024,), jnp.int32),
    ],
)


def _k2_body(m1, m2, raw, t3, lu3, wrow, btime, wtr, wtz, wtn, p_out):
  trel = (t3[0] - lu3[0]).astype(jnp.float32)
  tenc = jnp.cos(trel * wrow[...] + btime[...])
  a, b, r = m1[...], m2[...], raw[...]
  for g, wt in enumerate((wtr, wtz, wtn)):
    p_out[g] = (_mm(a, wt[0:64]) + _mm(b, wt[64:128])
                + _mm(r, wt[128:192]) + _mm(tenc, wt[192:224]))


def _k2(mem_rows, raw_msg, t3, lu3, wrow, btime, wtr, wtz, wtn):
  RB = 4096
  nb = B // RB
  grid = (nb, 2)
  return pl.pallas_call(
      _k2_body,
      grid=grid,
      in_specs=[
          pl.BlockSpec((RB, D), lambda i, h: (h * nb + i, 0)),
          pl.BlockSpec((RB, D), lambda i, h: ((1 - h) * nb + i, 0)),
          pl.BlockSpec((RB, D), lambda i, h: (i, 0)),
          pl.BlockSpec((1, RB, 1), lambda i, h: (i, 0, 0)),
          pl.BlockSpec((1, RB, 1), lambda i, h: (h * nb + i, 0, 0)),
          pl.BlockSpec((1, TD), lambda i, h: (0, 0)),
          pl.BlockSpec((1, TD), lambda i, h: (0, 0)),
          pl.BlockSpec((224, D), lambda i, h: (0, 0)),
          pl.BlockSpec((224, D), lambda i, h: (0, 0)),
          pl.BlockSpec((224, D), lambda i, h: (0, 0)),
      ],
      out_specs=pl.BlockSpec((3, RB, D), lambda i, h: (0, h * nb + i, 0)),
      out_shape=jax.ShapeDtypeStruct((3, TWO_B, D), jnp.float32),
  )(mem_rows, mem_rows, raw_msg, t3, lu3, wrow, btime, wtr, wtz, wtn)


def _k3_body(ids1d, p_parts, zrow, zblk, s_out, cnt1d,
             table, cnt_tbl, ids_v, pbuf, obuf, zscat, z1, ones_v, cbuf):
  c = lax.axis_index("c")
  s = lax.axis_index("s")
  r0 = s * 2048
  pltpu.sync_copy(ids1d.at[pl.ds(r0, 2048)], ids_v)
  pltpu.sync_copy(zblk, zscat)
  pltpu.sync_copy(zrow.at[0], z1)
  pltpu.sync_copy(zrow.at[1], ones_v)

  @pl.when(c == 0)
  def _counts():
    pltpu.sync_copy(z1, cnt_tbl.at[ids_v])
    plsc.subcore_barrier()
    pltpu.sync_copy(ones_v, cnt_tbl.at[ids_v], add=True)
    plsc.subcore_barrier()
    pltpu.sync_copy(cnt_tbl.at[ids_v], cbuf)
    pltpu.sync_copy(cbuf, cnt1d.at[pl.ds(r0, 2048)])

  for q in range(12):
    k = c * 12 + q
    part = k // 8
    col0 = (k % 8) * 8
    pltpu.sync_copy(p_parts.at[part, pl.ds(r0, 2048), pl.ds(col0, 8)], pbuf)
    pltpu.sync_copy(zscat, table.at[ids_v])
    plsc.subcore_barrier()
    pltpu.sync_copy(pbuf, table.at[ids_v], add=True)
    plsc.subcore_barrier()
    pltpu.sync_copy(table.at[ids_v], obuf)
    pltpu.sync_copy(obuf, s_out.at[part, pl.ds(r0, 2048), pl.ds(col0, 8)])
    plsc.subcore_barrier()


_k3 = pl.kernel(
    _k3_body,
    out_type=(
        jax.ShapeDtypeStruct((3, TWO_B, D), jnp.float32),
        jax.ShapeDtypeStruct((TWO_B,), jnp.float32),
    ),
    mesh=_MESH,
    compiler_params=_SC_PARAMS,
    scratch_types=[
        pltpu.VMEM_SHARED((N, 8), jnp.float32),
        pltpu.VMEM_SHARED((N,), jnp.float32),
        pltpu.VMEM((2048,), jnp.int32),
        pltpu.VMEM((2048, 8), jnp.float32),
        pltpu.VMEM((2048, 8), jnp.float32),
        pltpu.VMEM((2048, 8), jnp.float32),
        pltpu.VMEM((2048,), jnp.float32),
        pltpu.VMEM((2048,), jnp.float32),
        pltpu.VMEM((2048,), jnp.float32),
    ],
)


def _k4_body(s_parts, cnt3, mem_blk, whr, whz, whn,
             bir, biz, bin_, bhr, bhz, bhn, h_out):
  inv = 1.0 / cnt3[0]
  h = mem_blk[...]
  gi_r = s_parts[0] * inv + bir[...]
  gi_z = s_parts[1] * inv + biz[...]
  gi_n = s_parts[2] * inv + bin_[...]
  gh_r = _mm(h, whr[...]) + bhr[...]
  gh_z = _mm(h, whz[...]) + bhz[...]
  gh_n = _mm(h, whn[...]) + bhn[...]
  r = 1.0 / (1.0 + jnp.exp(-(gi_r + gh_r)))
  z = 1.0 / (1.0 + jnp.exp(-(gi_z + gh_z)))
  n = jnp.tanh(gi_n + r * gh_n)
  h_out[...] = (1.0 - z) * n + z * h


def _k4(s_parts, cnt3, mem_rows, whr, whz, whn, bir, biz, bin_, bhr, bhz, bhn):
  RB = 4096
  wspec = pl.BlockSpec((D, D), lambda i: (0, 0))
  bspec = pl.BlockSpec((1, D), lambda i: (0, 0))
  return pl.pallas_call(
      _k4_body,
      grid=(TWO_B // RB,),
      in_specs=[
          pl.BlockSpec((3, RB, D), lambda i: (0, i, 0)),
          pl.BlockSpec((1, RB, 1), lambda i: (i, 0, 0)),
          pl.BlockSpec((RB, D), lambda i: (i, 0)),
          wspec, wspec, wspec, bspec, bspec, bspec, bspec, bspec, bspec,
      ],
      out_specs=pl.BlockSpec((RB, D), lambda i: (i, 0)),
      out_shape=jax.ShapeDtypeStruct((TWO_B, D), jnp.float32),
  )(s_parts, cnt3, mem_rows, whr, whz, whn, bir, biz, bin_, bhr, bhz, bhn)


def _k5_body(ids1d, h_new, zrow, zmask_in, delta, mask0, mask1,
             idx_v, hbuf, zmask, ones_v):
  c = lax.axis_index("c")
  s = lax.axis_index("s")
  w = c * NS + s
  pltpu.sync_copy(zmask_in, zmask)
  pltpu.sync_copy(zrow.at[1, pl.ds(0, 1024)], ones_v)

  @pl.when(c == 0)
  def _z0():
    pltpu.sync_copy(zmask, mask0.at[pl.ds(s * 6256, 6256)])

  @pl.when(c == 1)
  def _z1():
    pltpu.sync_copy(zmask, mask1.at[pl.ds(s * 6256, 6256)])

  plsc.subcore_barrier()

  pltpu.sync_copy(ids1d.at[pl.ds(w * 1024, 1024)], idx_v)
  pltpu.sync_copy(h_new.at[pl.ds(w * 1024, 1024), :], hbuf)

  @pl.when(c == 0)
  def _m0():
    pltpu.sync_copy(ones_v, mask0.at[idx_v])

  @pl.when(c == 1)
  def _m1():
    pltpu.sync_copy(ones_v, mask1.at[idx_v])

  pltpu.sync_copy(hbuf, delta.at[idx_v])


_k5 = pl.kernel(
    _k5_body,
    out_type=(
        jax.ShapeDtypeStruct((N, D), jnp.float32),
        jax.ShapeDtypeStruct((MASK_PAD,), jnp.float32),
        jax.ShapeDtypeStruct((MASK_PAD,), jnp.float32),
    ),
    mesh=_MESH,
    compiler_params=_SC_PARAMS,
    scratch_types=[
        pltpu.VMEM((1024,), jnp.int32),
        pltpu.VMEM((1024, D), jnp.float32),
        pltpu.VMEM((6256,), jnp.float32),
        pltpu.VMEM((1024,), jnp.float32),
    ],
)


def _k6_body(memory, delta, m0, m1, out):
  touched = m0[0] + m1[0]
  out[...] = jnp.where(touched > 0.0, delta[...], memory[...])


def _k6(memory, delta, m0_3, m1_3):
  RB = 10000
  return pl.pallas_call(
      _k6_body,
      grid=(N // RB,),
      in_specs=[
          pl.BlockSpec((RB, D), lambda i: (i, 0)),
          pl.BlockSpec((RB, D), lambda i: (i, 0)),
          pl.BlockSpec((1, RB, 1), lambda i: (i, 0, 0)),
          pl.BlockSpec((1, RB, 1), lambda i: (i, 0, 0)),
      ],
      out_specs=pl.BlockSpec((RB, D), lambda i: (i, 0)),
      out_shape=jax.ShapeDtypeStruct((N, D), jnp.float32),
  )(memory, delta, m0_3, m1_3)


def kernel(src, pos_dst, neg_dst, t, raw_msg, memory, last_update,
           W_time, b_time, W_ih, W_hh, b_ih, b_hh):
  del neg_dst
  ids = jnp.concatenate([src, pos_dst]).astype(jnp.int32)

  mem_rows, lu1d = _k1(ids, memory, last_update.astype(jnp.int32))

  t3 = t.astype(jnp.int32).reshape(B // 4096, 4096, 1)
  lu3 = lu1d.reshape(TWO_B // 4096, 4096, 1)
  wrow = W_time[:, 0].reshape(1, TD)
  btime = b_time.reshape(1, TD)
  WT = W_ih.T
  wtr, wtz, wtn = WT[:, 0:D], WT[:, D:2 * D], WT[:, 2 * D:3 * D]
  p_parts = _k2(mem_rows, raw_msg, t3, lu3, wrow, btime, wtr, wtz, wtn)

  zrow = jnp.stack([jnp.zeros((2048,), jnp.float32),
                    jnp.ones((2048,), jnp.float32)])
  zblk = jnp.zeros((2048, 8), jnp.float32)
  s_parts, cnt1d = _k3(ids, p_parts, zrow, zblk)

  WHT = W_hh.T
  whr, whz, whn = WHT[:, 0:D], WHT[:, D:2 * D], WHT[:, 2 * D:3 * D]
  bir = b_ih[0:D].reshape(1, D)
  biz = b_ih[D:2 * D].reshape(1, D)
  bin_ = b_ih[2 * D:3 * D].reshape(1, D)
  bhr = b_hh[0:D].reshape(1, D)
  bhz = b_hh[D:2 * D].reshape(1, D)
  bhn = b_hh[2 * D:3 * D].reshape(1, D)
  cnt3 = cnt1d.reshape(TWO_B // 4096, 4096, 1)
  h_new = _k4(s_parts, cnt3, mem_rows, whr, whz, whn,
              bir, biz, bin_, bhr, bhz, bhn)

  zmask_in = jnp.zeros((6256,), jnp.float32)
  delta, mask0, mask1 = _k5(ids, h_new, zrow, zmask_in)
  m0_3 = mask0[:N].reshape(N // 10000, 10000, 1)
  m1_3 = mask1[:N].reshape(N // 10000, 10000, 1)
  return _k6(memory, delta, m0_3, m1_3)


# speedup vs baseline: 1.6341x; 1.6083x over previous
import jax
import jax.numpy as jnp
from jax import lax
from jax.experimental import pallas as pl
from jax.experimental.pallas import tpu as pltpu
from jax.experimental.pallas import tpu_sc as plsc

B = 16384
TWO_B = 2 * B
N = 100000
D = 64
TD = 32
NC = 2
NS = 16
NW = NC * NS

_MESH = plsc.VectorSubcoreMesh(core_axis_name="c", subcore_axis_name="s")
_SC_PARAMS = pltpu.CompilerParams(use_tc_tiling_on_sc=False)


def _mm(a, b):
  return lax.dot_general(a, b, (((1,), (0,)), ((), ())),
                         preferred_element_type=jnp.float32)


def _outer(a, b):
  return lax.dot_general(a, b, (((0,), (0,)), ((), ())),
                         precision=lax.Precision.HIGHEST,
                         preferred_element_type=jnp.float32)


def _k1_body(ids1d, mem_tbl, last_update, mem_rows, lu3, idx_v, rbuf, lubuf):
  c = lax.axis_index("c")
  s = lax.axis_index("s")
  w = c * NS + s
  pltpu.sync_copy(ids1d.at[pl.ds(w * 1024, 1024)], idx_v)
  pltpu.sync_copy(mem_tbl.at[idx_v], rbuf)
  pltpu.sync_copy(last_update.at[idx_v], lubuf)
  pltpu.sync_copy(rbuf, mem_rows.at[pl.ds(w * 1024, 1024), :])
  pltpu.sync_copy(lubuf, lu3.at[w // 4, 0, pl.ds((w % 4) * 1024, 1024)])


_k1 = pl.kernel(
    _k1_body,
    out_type=(
        jax.ShapeDtypeStruct((TWO_B, D), jnp.float32),
        jax.ShapeDtypeStruct((TWO_B // 4096, 1, 4096), jnp.int32),
    ),
    mesh=_MESH,
    compiler_params=_SC_PARAMS,
    scratch_types=[
        pltpu.VMEM((1024,), jnp.int32),
        pltpu.VMEM((1024, D), jnp.float32),
        pltpu.VMEM((1024,), jnp.int32),
    ],
)


def _k2_body(m1, m2, raw, t3, lu3, wrow, btime, wtr, wtz, wtn, p_out):
  trel = (t3[0] - lu3[0]).astype(jnp.float32)
  tenc = jnp.cos(_outer(trel, wrow[...]) + btime[...])
  a, b, r = m1[...], m2[...], raw[...]
  for g, wt in enumerate((wtr, wtz, wtn)):
    p_out[g] = (_mm(a, wt[0:64]) + _mm(b, wt[64:128])
                + _mm(r, wt[128:192]) + _mm(tenc, wt[192:224]))


def _k2(mem_rows, raw_msg, t3, lu3, wrow, btime, wtr, wtz, wtn):
  RB = 4096
  nb = B // RB
  grid = (nb, 2)
  return pl.pallas_call(
      _k2_body,
      grid=grid,
      in_specs=[
          pl.BlockSpec((RB, D), lambda i, h: (h * nb + i, 0)),
          pl.BlockSpec((RB, D), lambda i, h: ((1 - h) * nb + i, 0)),
          pl.BlockSpec((RB, D), lambda i, h: (i, 0)),
          pl.BlockSpec((1, 1, RB), lambda i, h: (i, 0, 0)),
          pl.BlockSpec((1, 1, RB), lambda i, h: (h * nb + i, 0, 0)),
          pl.BlockSpec((1, TD), lambda i, h: (0, 0)),
          pl.BlockSpec((1, TD), lambda i, h: (0, 0)),
          pl.BlockSpec((224, D), lambda i, h: (0, 0)),
          pl.BlockSpec((224, D), lambda i, h: (0, 0)),
          pl.BlockSpec((224, D), lambda i, h: (0, 0)),
      ],
      out_specs=pl.BlockSpec((3, RB, D), lambda i, h: (0, h * nb + i, 0)),
      out_shape=jax.ShapeDtypeStruct((3, TWO_B, D), jnp.float32),
  )(mem_rows, mem_rows, raw_msg, t3, lu3, wrow, btime, wtr, wtz, wtn)


def _k3_body(ids1d, p_parts, zrow, zblk, s_out, cnt3,
             table, cnt_tbl, ids_v, pbuf, obuf, zscat, z1, ones_v, cbuf):
  c = lax.axis_index("c")
  s = lax.axis_index("s")
  r0 = s * 2048
  pltpu.sync_copy(ids1d.at[pl.ds(r0, 2048)], ids_v)
  pltpu.sync_copy(zblk, zscat)
  pltpu.sync_copy(zrow.at[0], z1)
  pltpu.sync_copy(zrow.at[1], ones_v)

  @pl.when(c == 0)
  def _counts():
    pltpu.sync_copy(z1, cnt_tbl.at[ids_v])
    plsc.subcore_barrier()
    pltpu.sync_copy(ones_v, cnt_tbl.at[ids_v], add=True)
    plsc.subcore_barrier()
    pltpu.sync_copy(cnt_tbl.at[ids_v], cbuf)
    pltpu.sync_copy(cbuf, cnt3.at[s // 2, 0, pl.ds((s % 2) * 2048, 2048)])

  for q in range(12):
    k = c * 12 + q
    part = k // 8
    col0 = (k % 8) * 8
    pltpu.sync_copy(p_parts.at[part, pl.ds(r0, 2048), pl.ds(col0, 8)], pbuf)
    pltpu.sync_copy(zscat, table.at[ids_v])
    plsc.subcore_barrier()
    pltpu.sync_copy(pbuf, table.at[ids_v], add=True)
    plsc.subcore_barrier()
    pltpu.sync_copy(table.at[ids_v], obuf)
    pltpu.sync_copy(obuf, s_out.at[part, pl.ds(r0, 2048), pl.ds(col0, 8)])
    plsc.subcore_barrier()


_k3 = pl.kernel(
    _k3_body,
    out_type=(
        jax.ShapeDtypeStruct((3, TWO_B, D), jnp.float32),
        jax.ShapeDtypeStruct((TWO_B // 4096, 1, 4096), jnp.float32),
    ),
    mesh=_MESH,
    compiler_params=_SC_PARAMS,
    scratch_types=[
        pltpu.VMEM_SHARED((N, 8), jnp.float32),
        pltpu.VMEM_SHARED((N,), jnp.float32),
        pltpu.VMEM((2048,), jnp.int32),
        pltpu.VMEM((2048, 8), jnp.float32),
        pltpu.VMEM((2048, 8), jnp.float32),
        pltpu.VMEM((2048, 8), jnp.float32),
        pltpu.VMEM((2048,), jnp.float32),
        pltpu.VMEM((2048,), jnp.float32),
        pltpu.VMEM((2048,), jnp.float32),
    ],
)


def _k4_body(s_parts, cnt3, mem_blk, whr, whz, whn,
             bir, biz, bin_, bhr, bhz, bhn, h_out):
  ones = jnp.ones((1, D), jnp.float32)
  inv = _outer(1.0 / cnt3[0], ones)
  h = mem_blk[...]
  gi_r = s_parts[0] * inv + bir[...]
  gi_z = s_parts[1] * inv + biz[...]
  gi_n = s_parts[2] * inv + bin_[...]
  gh_r = _mm(h, whr[...]) + bhr[...]
  gh_z = _mm(h, whz[...]) + bhz[...]
  gh_n = _mm(h, whn[...]) + bhn[...]
  r = 1.0 / (1.0 + jnp.exp(-(gi_r + gh_r)))
  z = 1.0 / (1.0 + jnp.exp(-(gi_z + gh_z)))
  n = jnp.tanh(gi_n + r * gh_n)
  h_out[...] = (1.0 - z) * n + z * h


def _k4(s_parts, cnt3, mem_rows, whr, whz, whn, bir, biz, bin_, bhr, bhz, bhn):
  RB = 4096
  wspec = pl.BlockSpec((D, D), lambda i: (0, 0))
  bspec = pl.BlockSpec((1, D), lambda i: (0, 0))
  return pl.pallas_call(
      _k4_body,
      grid=(TWO_B // RB,),
      in_specs=[
          pl.BlockSpec((3, RB, D), lambda i: (0, i, 0)),
          pl.BlockSpec((1, 1, RB), lambda i: (i, 0, 0)),
          pl.BlockSpec((RB, D), lambda i: (i, 0)),
          wspec, wspec, wspec, bspec, bspec, bspec, bspec, bspec, bspec,
      ],
      out_specs=pl.BlockSpec((RB, D), lambda i: (i, 0)),
      out_shape=jax.ShapeDtypeStruct((TWO_B, D), jnp.float32),
  )(s_parts, cnt3, mem_rows, whr, whz, whn, bir, biz, bin_, bhr, bhz, bhn)


def _k5_body(ids1d, h_new, out_tbl, idx_v, hbuf):
  c = lax.axis_index("c")
  s = lax.axis_index("s")
  w = c * NS + s
  pltpu.sync_copy(ids1d.at[pl.ds(w * 1024, 1024)], idx_v)
  pltpu.sync_copy(h_new.at[pl.ds(w * 1024, 1024), :], hbuf)
  pltpu.sync_copy(hbuf, out_tbl.at[idx_v])


_k5 = pl.kernel(
    _k5_body,
    out_type=(),
    mesh=_MESH,
    compiler_params=_SC_PARAMS,
    scratch_types=[
        pltpu.VMEM((1024,), jnp.int32),
        pltpu.VMEM((1024, D), jnp.float32),
    ],
)


def kernel(src, pos_dst, neg_dst, t, raw_msg, memory, last_update,
           W_time, b_time, W_ih, W_hh, b_ih, b_hh):
  del neg_dst
  ids = jnp.concatenate([src, pos_dst]).astype(jnp.int32)
  mem_ref = jax.new_ref(memory)

  mem_rows, lu3 = _k1(ids, mem_ref, last_update.astype(jnp.int32))

  t3 = t.astype(jnp.int32).reshape(B // 4096, 1, 4096)
  wrow = W_time[:, 0].reshape(1, TD)
  btime = b_time.reshape(1, TD)
  WT = W_ih.T
  wtr, wtz, wtn = WT[:, 0:D], WT[:, D:2 * D], WT[:, 2 * D:3 * D]
  p_parts = _k2(mem_rows, raw_msg, t3, lu3, wrow, btime, wtr, wtz, wtn)

  zrow = jnp.stack([jnp.zeros((2048,), jnp.float32),
                    jnp.ones((2048,), jnp.float32)])
  zblk = jnp.zeros((2048, 8), jnp.float32)
  s_parts, cnt3 = _k3(ids, p_parts, zrow, zblk)

  WHT = W_hh.T
  whr, whz, whn = WHT[:, 0:D], WHT[:, D:2 * D], WHT[:, 2 * D:3 * D]
  bir = b_ih[0:D].reshape(1, D)
  biz = b_ih[D:2 * D].reshape(1, D)
  bin_ = b_ih[2 * D:3 * D].reshape(1, D)
  bhr = b_hh[0:D].reshape(1, D)
  bhz = b_hh[D:2 * D].reshape(1, D)
  bhn = b_hh[2 * D:3 * D].reshape(1, D)
  h_new = _k4(s_parts, cnt3, mem_rows, whr, whz, whn,
              bir, biz, bin_, bhr, bhz, bhn)

  _k5(ids, h_new, mem_ref)
  return mem_ref[...]


# speedup vs baseline: 1.7344x; 1.0614x over previous
import jax
import jax.numpy as jnp
from jax import lax
from jax.experimental import pallas as pl
from jax.experimental.pallas import tpu as pltpu
from jax.experimental.pallas import tpu_sc as plsc

B = 16384
TWO_B = 2 * B
N = 100000
D = 64
TD = 32
NC = 2
NS = 16
NW = NC * NS

_MESH = plsc.VectorSubcoreMesh(core_axis_name="c", subcore_axis_name="s")
_SC_PARAMS = pltpu.CompilerParams(use_tc_tiling_on_sc=False)


def _mm(a, b):
  return lax.dot_general(a, b, (((1,), (0,)), ((), ())),
                         preferred_element_type=jnp.float32)


def _outer(a, b):
  return lax.dot_general(a, b, (((0,), (0,)), ((), ())),
                         precision=lax.Precision.HIGHEST,
                         preferred_element_type=jnp.float32)


def _k1_body(ids1d, mem_tbl, last_update, mem_rows, lu3, idx_v, rbuf, lubuf):
  c = lax.axis_index("c")
  s = lax.axis_index("s")
  w = c * NS + s
  pltpu.sync_copy(ids1d.at[pl.ds(w * 1024, 1024)], idx_v)
  pltpu.sync_copy(mem_tbl.at[idx_v], rbuf)
  pltpu.sync_copy(last_update.at[idx_v], lubuf)
  pltpu.sync_copy(rbuf, mem_rows.at[pl.ds(w * 1024, 1024), :])
  pltpu.sync_copy(lubuf, lu3.at[w // 4, 0, pl.ds((w % 4) * 1024, 1024)])


_k1 = pl.kernel(
    _k1_body,
    out_type=(
        jax.ShapeDtypeStruct((TWO_B, D), jnp.float32),
        jax.ShapeDtypeStruct((TWO_B // 4096, 1, 4096), jnp.int32),
    ),
    mesh=_MESH,
    compiler_params=_SC_PARAMS,
    scratch_types=[
        pltpu.VMEM((1024,), jnp.int32),
        pltpu.VMEM((1024, D), jnp.float32),
        pltpu.VMEM((1024,), jnp.int32),
    ],
)


def _k2_body(m1, m2, raw, t3, lu3, wrow, btime, wt, p_out):
  trel = (t3[0] - lu3[0]).astype(jnp.float32)
  tenc = jnp.cos(_outer(trel, wrow[...]) + btime[...])
  x = jnp.concatenate([m1[...], m2[...], raw[...], tenc], axis=1)
  p_out[...] = _mm(x, wt[...])


def _k2(mem_rows, raw_msg, t3, lu3, wrow, btime, wt):
  RB = 4096
  nb = B // RB
  grid = (nb, 2)
  return pl.pallas_call(
      _k2_body,
      grid=grid,
      in_specs=[
          pl.BlockSpec((RB, D), lambda i, h: (h * nb + i, 0)),
          pl.BlockSpec((RB, D), lambda i, h: ((1 - h) * nb + i, 0)),
          pl.BlockSpec((RB, D), lambda i, h: (i, 0)),
          pl.BlockSpec((1, 1, RB), lambda i, h: (i, 0, 0)),
          pl.BlockSpec((1, 1, RB), lambda i, h: (h * nb + i, 0, 0)),
          pl.BlockSpec((1, TD), lambda i, h: (0, 0)),
          pl.BlockSpec((1, TD), lambda i, h: (0, 0)),
          pl.BlockSpec((224, 192), lambda i, h: (0, 0)),
      ],
      out_specs=pl.BlockSpec((RB, 192), lambda i, h: (h * nb + i, 0)),
      out_shape=jax.ShapeDtypeStruct((TWO_B, 192), jnp.float32),
  )(mem_rows, mem_rows, raw_msg, t3, lu3, wrow, btime, wt)


def _k3_body(ids1d, p_parts, zrow, zblk, s_out, cnt3,
             table, cnt_tbl, ids_v, pbuf, obuf, zscat, z1, ones_v, cbuf):
  c = lax.axis_index("c")
  s = lax.axis_index("s")
  r0 = s * 2048
  pltpu.sync_copy(ids1d.at[pl.ds(r0, 2048)], ids_v)
  pltpu.sync_copy(zblk, zscat)
  pltpu.sync_copy(zrow.at[0], z1)
  pltpu.sync_copy(zrow.at[1], ones_v)

  @pl.when(c == 0)
  def _counts():
    pltpu.sync_copy(z1, cnt_tbl.at[ids_v])
    plsc.subcore_barrier()
    pltpu.sync_copy(ones_v, cnt_tbl.at[ids_v], add=True)
    plsc.subcore_barrier()
    pltpu.sync_copy(cnt_tbl.at[ids_v], cbuf)
    pltpu.sync_copy(cbuf, cnt3.at[s // 2, 0, pl.ds((s % 2) * 2048, 2048)])

  for q in range(12):
    col0 = c * 96 + q * 8
    pltpu.sync_copy(p_parts.at[pl.ds(r0, 2048), pl.ds(col0, 8)], pbuf)
    pltpu.sync_copy(zscat, table.at[ids_v])
    plsc.subcore_barrier()
    pltpu.sync_copy(pbuf, table.at[ids_v], add=True)
    plsc.subcore_barrier()
    pltpu.sync_copy(table.at[ids_v], obuf)
    pltpu.sync_copy(obuf, s_out.at[pl.ds(r0, 2048), pl.ds(col0, 8)])
    plsc.subcore_barrier()


_k3 = pl.kernel(
    _k3_body,
    out_type=(
        jax.ShapeDtypeStruct((TWO_B, 192), jnp.float32),
        jax.ShapeDtypeStruct((TWO_B // 4096, 1, 4096), jnp.float32),
    ),
    mesh=_MESH,
    compiler_params=_SC_PARAMS,
    scratch_types=[
        pltpu.VMEM_SHARED((N, 8), jnp.float32),
        pltpu.VMEM_SHARED((N,), jnp.float32),
        pltpu.VMEM((2048,), jnp.int32),
        pltpu.VMEM((2048, 8), jnp.float32),
        pltpu.VMEM((2048, 8), jnp.float32),
        pltpu.VMEM((2048, 8), jnp.float32),
        pltpu.VMEM((2048,), jnp.float32),
        pltpu.VMEM((2048,), jnp.float32),
        pltpu.VMEM((2048,), jnp.float32),
    ],
)


def _k4_body(s_blk, cnt3, mem_blk, wh, bi, bh, h_out):
  ones = jnp.ones((1, 192), jnp.float32)
  inv = _outer(1.0 / cnt3[0], ones)
  h = mem_blk[...]
  gi = s_blk[...] * inv + bi[...]
  gh = _mm(h, wh[...]) + bh[...]
  x = gi + gh
  r = 1.0 / (1.0 + jnp.exp(-x[:, 0:64]))
  z = 1.0 / (1.0 + jnp.exp(-x[:, 64:128]))
  n = jnp.tanh(gi[:, 128:192] + r * gh[:, 128:192])
  h_out[...] = (1.0 - z) * n + z * h


def _k4(s_flat, cnt3, mem_rows, wh, bi, bh):
  RB = 4096
  return pl.pallas_call(
      _k4_body,
      grid=(TWO_B // RB,),
      in_specs=[
          pl.BlockSpec((RB, 192), lambda i: (i, 0)),
          pl.BlockSpec((1, 1, RB), lambda i: (i, 0, 0)),
          pl.BlockSpec((RB, D), lambda i: (i, 0)),
          pl.BlockSpec((D, 192), lambda i: (0, 0)),
          pl.BlockSpec((1, 192), lambda i: (0, 0)),
          pl.BlockSpec((1, 192), lambda i: (0, 0)),
      ],
      out_specs=pl.BlockSpec((RB, D), lambda i: (i, 0)),
      out_shape=jax.ShapeDtypeStruct((TWO_B, D), jnp.float32),
  )(s_flat, cnt3, mem_rows, wh, bi, bh)


def _k5_body(ids1d, h_new, out_tbl, idx_v, hbuf):
  c = lax.axis_index("c")
  s = lax.axis_index("s")
  w = c * NS + s
  pltpu.sync_copy(ids1d.at[pl.ds(w * 1024, 1024)], idx_v)
  pltpu.sync_copy(h_new.at[pl.ds(w * 1024, 1024), :], hbuf)
  pltpu.sync_copy(hbuf, out_tbl.at[idx_v])


_k5 = pl.kernel(
    _k5_body,
    out_type=(),
    mesh=_MESH,
    compiler_params=_SC_PARAMS,
    scratch_types=[
        pltpu.VMEM((1024,), jnp.int32),
        pltpu.VMEM((1024, D), jnp.float32),
    ],
)


def kernel(src, pos_dst, neg_dst, t, raw_msg, memory, last_update,
           W_time, b_time, W_ih, W_hh, b_ih, b_hh):
  del neg_dst
  ids = jnp.concatenate([src, pos_dst]).astype(jnp.int32)
  mem_ref = jax.new_ref(memory)

  mem_rows, lu3 = _k1(ids, mem_ref, last_update.astype(jnp.int32))

  t3 = t.astype(jnp.int32).reshape(B // 4096, 1, 4096)
  wrow = W_time[:, 0].reshape(1, TD)
  btime = b_time.reshape(1, TD)
  WT = W_ih.T
  p_parts = _k2(mem_rows, raw_msg, t3, lu3, wrow, btime, WT)

  zrow = jnp.stack([jnp.zeros((2048,), jnp.float32),
                    jnp.ones((2048,), jnp.float32)])
  zblk = jnp.zeros((2048, 8), jnp.float32)
  s_parts, cnt3 = _k3(ids, p_parts, zrow, zblk)

  WHT = W_hh.T
  h_new = _k4(s_parts, cnt3, mem_rows, WHT,
              b_ih.reshape(1, 192), b_hh.reshape(1, 192))

  _k5(ids, h_new, mem_ref)
  return mem_ref[...]
